# baseline (speedup 1.0000x reference)
import functools

import jax
import jax.numpy as jnp
from jax import lax
from jax.experimental import pallas as pl
from jax.experimental.pallas import tpu as pltpu
from jax.experimental.pallas import tpu_sc as plsc

VOCAB = 1000000
EMBED_DIM = 64
PROJ_DIM = 32
FEAT = 10
OUT_DIM = 96
B = 4096
L = 200
BL = B * L

NUM_CORES = 2
NUM_SUBCORES = 16
NW = NUM_CORES * NUM_SUBCORES
CB = 256
NBQ = B // CB
NCELL = L * NBQ
CPW = NCELL // NW
NSUB = CB // 16

_mesh = plsc.VectorSubcoreMesh(core_axis_name="c", subcore_axis_name="s")


@functools.partial(
    pl.kernel,
    mesh=_mesh,
    compiler_params=pltpu.CompilerParams(
        needs_layout_passes=False, use_tc_tiling_on_sc=False),
    out_type=jax.ShapeDtypeStruct((OUT_DIM, L * B), jnp.float32),
    scratch_types=[
        pltpu.VMEM((2, FEAT, CB), jnp.float32),
        pltpu.VMEM((2, CB), jnp.int32),
        pltpu.VMEM((2, CB, EMBED_DIM), jnp.float32),
        pltpu.VMEM((2, 112, CB), jnp.float32),
        pltpu.VMEM((PROJ_DIM * (FEAT - 1) * 16,), jnp.float32),
        pltpu.SemaphoreType.DMA((2,)),
        pltpu.SemaphoreType.DMA((2,)),
    ],
)
def _encode(x_hbm, tab_hbm, ws_hbm, out_hbm, xv, idv, rows, buf, wv,
            gsem, osem):
    wid = lax.axis_index("s") * NUM_CORES + lax.axis_index("c")
    pltpu.sync_copy(ws_hbm, wv)
    biota = lax.iota(jnp.int32, 16)

    def cell_lb(c):
        cell = wid * CPW + c
        return cell // NBQ, (cell % NBQ) * CB

    def stage1(c, k):
        l, b0 = cell_lb(c)
        pltpu.sync_copy(
            x_hbm.at[pl.ds(0, FEAT), pl.ds(l * B + b0, CB)], xv.at[k])

        def id_body(j, _):
            idv[k, pl.ds(j * 16, 16)] = (
                xv[k, 0, pl.ds(j * 16, 16)].astype(jnp.int32))
            return 0
        lax.fori_loop(0, NSUB, id_body, 0)
        pltpu.make_async_copy(
            tab_hbm.at[idv.at[k]], rows.at[k], gsem.at[k]).start()

    def wait_gather(k):
        pltpu.make_async_copy(
            tab_hbm.at[idv.at[k]], rows.at[k], gsem.at[k]).wait()

    def wait_out(k):
        pltpu.make_async_copy(
            buf.at[k, pl.ds(0, EMBED_DIM)],
            out_hbm.at[pl.ds(0, EMBED_DIM), pl.ds(0, CB)],
            osem.at[k]).wait()
        pltpu.make_async_copy(
            buf.at[k, pl.ds(80, PROJ_DIM)],
            out_hbm.at[pl.ds(0, PROJ_DIM), pl.ds(0, CB)],
            osem.at[k]).wait()

    def compute(c, k):
        def emb_body(s, _):
            bidx = s * 16 + biota
            for col in range(EMBED_DIM):
                v = plsc.load_gather(
                    rows.at[k], [bidx, jnp.full((16,), col, jnp.int32)])
                buf[k, col, pl.ds(s * 16, 16)] = v
            return 0
        lax.fori_loop(0, NSUB, emb_body, 0)

        for cg in reversed(range(8)):

            def proj_body(s, _):
                feats = [xv[k, f + 1, pl.ds(s * 16, 16)]
                         for f in range(FEAT - 1)]
                for cc in range(4):
                    acc = None
                    for f in range(FEAT - 1):
                        w = wv[pl.ds(((cg * 4 + cc) * (FEAT - 1) + f) * 16,
                                     16)]
                        t = feats[f] * w
                        acc = t if acc is None else acc + t
                    buf[k, 80 + cg * 4 + cc, pl.ds(s * 16, 16)] = acc
                return 0
            lax.fori_loop(0, NSUB, proj_body, 0)

    def fire_out(c, k):
        l, b0 = cell_lb(c)
        pltpu.make_async_copy(
            buf.at[k, pl.ds(0, EMBED_DIM)],
            out_hbm.at[pl.ds(0, EMBED_DIM), pl.ds(l * B + b0, CB)],
            osem.at[k]).start()
        pltpu.make_async_copy(
            buf.at[k, pl.ds(80, PROJ_DIM)],
            out_hbm.at[pl.ds(EMBED_DIM, PROJ_DIM), pl.ds(l * B + b0, CB)],
            osem.at[k]).start()

    stage1(0, 0)

    stage1(1, 1)
    wait_gather(0)
    compute(0, 0)
    fire_out(0, 0)

    def pair_body(j, _):
        for p in range(2):
            c = 2 * j + 1 + p
            k = (1 + p) % 2
            wait_out(k ^ 1)
            stage1(c + 1, k ^ 1)
            wait_gather(k)
            compute(c, k)
            fire_out(c, k)
        return 0
    lax.fori_loop(0, (CPW - 2) // 2, pair_body, 0)

    wait_out(0)
    wait_gather(1)
    compute(CPW - 1, 1)
    fire_out(CPW - 1, 1)
    wait_out(1)


def kernel(x, table, W):
    xt = x.transpose(2, 1, 0).reshape(FEAT, L * B)
    ws = jnp.repeat((W / 255.0).reshape(-1), 16).astype(jnp.float32)
    out = _encode(xt, table, ws)
    return out.reshape(OUT_DIM, L, B).transpose(2, 1, 0)


# speedup vs baseline: 1.0671x; 1.0671x over previous
import functools

import jax
import jax.numpy as jnp
from jax import lax
from jax.experimental import pallas as pl
from jax.experimental.pallas import tpu as pltpu
from jax.experimental.pallas import tpu_sc as plsc

VOCAB = 1000000
EMBED_DIM = 64
PROJ_DIM = 32
FEAT = 10
OUT_DIM = 96
B = 4096
L = 200
BL = B * L

NUM_CORES = 2
NUM_SUBCORES = 16
NW = NUM_CORES * NUM_SUBCORES
CB = 256
NBQ = B // CB
NCELL = L * NBQ
CPW = NCELL // NW
NSUB = CB // 16

_mesh = plsc.VectorSubcoreMesh(core_axis_name="c", subcore_axis_name="s")


@functools.partial(
    pl.kernel,
    mesh=_mesh,
    compiler_params=pltpu.CompilerParams(
        needs_layout_passes=False, use_tc_tiling_on_sc=False),
    out_type=jax.ShapeDtypeStruct((L * OUT_DIM, B), jnp.float32),
    scratch_types=[
        pltpu.VMEM((2, FEAT, CB), jnp.float32),
        pltpu.VMEM((2, CB), jnp.int32),
        pltpu.VMEM((2, CB, EMBED_DIM), jnp.float32),
        pltpu.VMEM((2, 112, CB), jnp.float32),
        pltpu.VMEM((PROJ_DIM * (FEAT - 1) * 16,), jnp.float32),
        pltpu.SemaphoreType.DMA((2,)),
        pltpu.SemaphoreType.DMA((2,)),
    ],
)
def _encode(x_hbm, tab_hbm, ws_hbm, out_hbm, xv, idv, rows, buf, wv,
            gsem, osem):
    wid = lax.axis_index("s") * NUM_CORES + lax.axis_index("c")
    pltpu.sync_copy(ws_hbm, wv)
    biota = lax.iota(jnp.int32, 16)

    def cell_lb(c):
        cell = wid * CPW + c
        return cell // NBQ, (cell % NBQ) * CB

    def stage1(c, k):
        l, b0 = cell_lb(c)
        pltpu.sync_copy(
            x_hbm.at[pl.ds(0, FEAT), pl.ds(l * B + b0, CB)], xv.at[k])

        def id_body(j, _):
            idv[k, pl.ds(j * 16, 16)] = (
                xv[k, 0, pl.ds(j * 16, 16)].astype(jnp.int32))
            return 0
        lax.fori_loop(0, NSUB, id_body, 0)
        pltpu.make_async_copy(
            tab_hbm.at[idv.at[k]], rows.at[k], gsem.at[k]).start()

    def wait_gather(k):
        pltpu.make_async_copy(
            tab_hbm.at[idv.at[k]], rows.at[k], gsem.at[k]).wait()

    def wait_out(k):
        pltpu.make_async_copy(
            buf.at[k, pl.ds(0, EMBED_DIM)],
            out_hbm.at[pl.ds(0, EMBED_DIM), pl.ds(0, CB)],
            osem.at[k]).wait()
        pltpu.make_async_copy(
            buf.at[k, pl.ds(80, PROJ_DIM)],
            out_hbm.at[pl.ds(0, PROJ_DIM), pl.ds(0, CB)],
            osem.at[k]).wait()

    def compute(c, k):
        def emb_body(s, _):
            bidx = s * 16 + biota
            for col in range(EMBED_DIM):
                v = plsc.load_gather(
                    rows.at[k], [bidx, jnp.full((16,), col, jnp.int32)])
                buf[k, col, pl.ds(s * 16, 16)] = v
            return 0
        lax.fori_loop(0, NSUB, emb_body, 0)

        for cg in reversed(range(8)):

            def proj_body(s, _):
                feats = [xv[k, f + 1, pl.ds(s * 16, 16)]
                         for f in range(FEAT - 1)]
                for cc in range(4):
                    acc = None
                    for f in range(FEAT - 1):
                        w = wv[pl.ds(((cg * 4 + cc) * (FEAT - 1) + f) * 16,
                                     16)]
                        t = feats[f] * w
                        acc = t if acc is None else acc + t
                    buf[k, 80 + cg * 4 + cc, pl.ds(s * 16, 16)] = acc
                return 0
            lax.fori_loop(0, NSUB, proj_body, 0)

    def fire_out(c, k):
        l, b0 = cell_lb(c)
        pltpu.make_async_copy(
            buf.at[k, pl.ds(0, EMBED_DIM)],
            out_hbm.at[pl.ds(l * OUT_DIM, EMBED_DIM), pl.ds(b0, CB)],
            osem.at[k]).start()
        pltpu.make_async_copy(
            buf.at[k, pl.ds(80, PROJ_DIM)],
            out_hbm.at[pl.ds(l * OUT_DIM + EMBED_DIM, PROJ_DIM),
                       pl.ds(b0, CB)],
            osem.at[k]).start()

    stage1(0, 0)

    stage1(1, 1)
    wait_gather(0)
    compute(0, 0)
    fire_out(0, 0)

    def pair_body(j, _):
        for p in range(2):
            c = 2 * j + 1 + p
            k = (1 + p) % 2
            wait_out(k ^ 1)
            stage1(c + 1, k ^ 1)
            wait_gather(k)
            compute(c, k)
            fire_out(c, k)
        return 0
    lax.fori_loop(0, (CPW - 2) // 2, pair_body, 0)

    wait_out(0)
    wait_gather(1)
    compute(CPW - 1, 1)
    fire_out(CPW - 1, 1)
    wait_out(1)


def kernel(x, table, W):
    xt = x.transpose(2, 1, 0).reshape(FEAT, L * B)
    ws = jnp.repeat((W / 255.0).reshape(-1), 16).astype(jnp.float32)
    out = _encode(xt, table, ws)
    return out.reshape(L, OUT_DIM, B).transpose(2, 0, 1)


# speedup vs baseline: 1.5569x; 1.4590x over previous
import functools

import jax
import jax.numpy as jnp
from jax import lax
from jax.experimental import pallas as pl
from jax.experimental.pallas import tpu as pltpu
from jax.experimental.pallas import tpu_sc as plsc

VOCAB = 1000000
EMBED_DIM = 64
PROJ_DIM = 32
FEAT = 10
OUT_DIM = 96
B = 4096
L = 200
BL = B * L

NUM_CORES = 2
NUM_SUBCORES = 16
NW = NUM_CORES * NUM_SUBCORES
ITEMS_PER_W = BL // NW
CB = 512
NCHUNK = ITEMS_PER_W // CB

_mesh = plsc.VectorSubcoreMesh(core_axis_name="c", subcore_axis_name="s")


@functools.partial(
    pl.kernel,
    mesh=_mesh,
    compiler_params=pltpu.CompilerParams(
        needs_layout_passes=False, use_tc_tiling_on_sc=False),
    out_type=jax.ShapeDtypeStruct((BL, OUT_DIM), jnp.float32),
    scratch_types=[
        pltpu.VMEM((2, CB * FEAT), jnp.float32),
        pltpu.VMEM((2, CB), jnp.int32),
        pltpu.VMEM((2, CB, EMBED_DIM), jnp.float32),
        pltpu.VMEM((2, CB, PROJ_DIM), jnp.float32),
        pltpu.VMEM(((FEAT - 1) * PROJ_DIM,), jnp.float32),
        pltpu.SemaphoreType.DMA((2,)),
        pltpu.SemaphoreType.DMA((2,)),
        pltpu.SemaphoreType.DMA((2,)),
    ],
)
def _encode(x_hbm, tab_hbm, ws_hbm, out_hbm,
            xv, idv, rows, projv, wv, gsem, rsem, psem):
    wid = lax.axis_index("s") * NUM_CORES + lax.axis_index("c")
    pltpu.sync_copy(ws_hbm, wv)
    lane10 = lax.iota(jnp.int32, 16) * FEAT

    wregs = [wv[pl.ds(k * PROJ_DIM + h * 16, 16)]
             for k in range(FEAT - 1) for h in range(2)]

    def chunk_base(c):
        return wid * ITEMS_PER_W + c * CB

    def stage1(c, k):
        base = chunk_base(c)
        pltpu.sync_copy(x_hbm.at[pl.ds(base * FEAT, CB * FEAT)], xv.at[k])

        def id_body(j, _):
            idxs = j * (16 * FEAT) + lane10
            idv[k, pl.ds(j * 16, 16)] = (
                plsc.load_gather(xv.at[k], [idxs]).astype(jnp.int32))
            return 0
        lax.fori_loop(0, CB // 16, id_body, 0)
        pltpu.make_async_copy(
            tab_hbm.at[idv.at[k]], rows.at[k], gsem.at[k]).start()

    def wait_gather(k):
        pltpu.make_async_copy(
            tab_hbm.at[idv.at[k]], rows.at[k], gsem.at[k]).wait()

    def wait_outs(k):
        pltpu.make_async_copy(
            rows.at[k],
            out_hbm.at[pl.ds(0, CB), pl.ds(0, EMBED_DIM)],
            rsem.at[k]).wait()
        pltpu.make_async_copy(
            projv.at[k],
            out_hbm.at[pl.ds(0, CB), pl.ds(EMBED_DIM, PROJ_DIM)],
            psem.at[k]).wait()

    def compute(c, k):
        def item_body(i, _):
            accs = [None, None]
            for f in range(FEAT - 1):
                s = plsc.load_gather(
                    xv.at[k], [jnp.full((16,), i * FEAT + 1 + f, jnp.int32)])
                for h in range(2):
                    t = s * wregs[2 * f + h]
                    accs[h] = t if accs[h] is None else accs[h] + t
            for h in range(2):
                projv[k, i, pl.ds(h * 16, 16)] = accs[h]
            return 0
        lax.fori_loop(0, CB, item_body, 0, unroll=8)

    def fire_outs(c, k):
        base = chunk_base(c)
        pltpu.make_async_copy(
            rows.at[k],
            out_hbm.at[pl.ds(base, CB), pl.ds(0, EMBED_DIM)],
            rsem.at[k]).start()
        pltpu.make_async_copy(
            projv.at[k],
            out_hbm.at[pl.ds(base, CB), pl.ds(EMBED_DIM, PROJ_DIM)],
            psem.at[k]).start()

    stage1(0, 0)

    stage1(1, 1)
    wait_gather(0)
    compute(0, 0)
    fire_outs(0, 0)

    def pair_body(j, _):
        for p in range(2):
            c = 2 * j + 1 + p
            k = (1 + p) % 2
            wait_outs(k ^ 1)
            stage1(c + 1, k ^ 1)
            wait_gather(k)
            compute(c, k)
            fire_outs(c, k)
        return 0
    lax.fori_loop(0, (NCHUNK - 2) // 2, pair_body, 0)

    wait_outs(0)
    wait_gather(1)
    compute(NCHUNK - 1, 1)
    fire_outs(NCHUNK - 1, 1)
    wait_outs(1)


def kernel(x, table, W):
    x_flat = x.reshape(BL * FEAT)
    ws = (W.T / 255.0).reshape((FEAT - 1) * PROJ_DIM).astype(jnp.float32)
    out = _encode(x_flat, table, ws)
    return out.reshape(B, L, OUT_DIM)
